# baseline (speedup 1.0000x reference)
import functools

import jax
import jax.numpy as jnp
from jax import lax
from jax.experimental import pallas as pl
from jax.experimental.pallas import tpu as pltpu
from jax.experimental.pallas import tpu_sc as plsc

VOCAB = 1000000
D = 64
B = 16384 * 50
NC, NS, L = 2, 16, 16
NW = NC * NS
B_PER_W = B // NW
C = 128
NCHUNK = B_PER_W // C
HALF = NCHUNK // 2


def _body(table_hbm, idx_hbm, scale_hbm, out_hbm,
          idx_v, scale_v, buf0, buf1, sem0, sem1):
    wid = lax.axis_index("s") * NC + lax.axis_index("c")
    base = wid * B_PER_W

    pltpu.sync_copy(idx_hbm.at[wid], idx_v)
    pltpu.sync_copy(scale_hbm, scale_v)
    svec = scale_v[...]

    def gather(c, buf, sem):
        pltpu.async_copy(table_hbm.at[idx_v.at[c]], buf, sem)

    def wait(buf, sem):
        pltpu.make_async_copy(table_hbm.at[idx_v.at[0]], buf, sem).wait()

    def scale_and_store(c, buf):
        def row(i, _):
            for k in range(D // L):
                sl = pl.ds(k * L, L)
                buf[i, sl] = buf[i, sl] * svec
            return 0
        lax.fori_loop(0, C, row, 0)
        pltpu.sync_copy(buf, out_hbm.at[pl.ds(base + c * C, C)])

    gather(0, buf0, sem0)

    def step(t, _):
        c0 = 2 * t
        gather(c0 + 1, buf1, sem1)
        wait(buf0, sem0)
        scale_and_store(c0, buf0)

        @pl.when(t < HALF - 1)
        def _():
            gather(c0 + 2, buf0, sem0)

        wait(buf1, sem1)
        scale_and_store(c0 + 1, buf1)
        return 0

    lax.fori_loop(0, HALF, step, 0)


@jax.jit
def _embed(table, idx, scale_vec):
    mesh = plsc.VectorSubcoreMesh(core_axis_name="c", subcore_axis_name="s")
    k = pl.kernel(
        _body,
        out_type=jax.ShapeDtypeStruct((B, D), jnp.float32),
        mesh=mesh,
        scratch_types=[
            pltpu.VMEM((NCHUNK, C), jnp.int32),
            pltpu.VMEM((L,), jnp.float32),
            pltpu.VMEM((C, D), jnp.float32),
            pltpu.VMEM((C, D), jnp.float32),
            pltpu.SemaphoreType.DMA,
            pltpu.SemaphoreType.DMA,
        ],
        compiler_params=pltpu.CompilerParams(use_tc_tiling_on_sc=False),
    )
    return k(table, idx, scale_vec)


def kernel(token_ids, embed_weight, scale):
    orig_shape = token_ids.shape
    idx = token_ids.reshape(NW, NCHUNK, C).astype(jnp.int32)
    scale_vec = jnp.broadcast_to(scale.astype(jnp.float32), (L,))
    out = _embed(embed_weight, idx, scale_vec)
    return out.reshape(*orig_shape, D)


# speedup vs baseline: 1.0533x; 1.0533x over previous
import jax
import jax.numpy as jnp
from jax import lax
from jax.experimental import pallas as pl
from jax.experimental.pallas import tpu as pltpu
from jax.experimental.pallas import tpu_sc as plsc

VOCAB = 1000000
D = 64
B = 16384 * 50
NC, NS, L = 2, 16, 16
NW = NC * NS
B_PER_W = B // NW
C = 128
NCHUNK = B_PER_W // C
PAIR = 2 * C
NPAIR = NCHUNK // 2
NBUF = 5
ROUNDS = NPAIR // NBUF


def _body(table_hbm, idx_hbm, scale_hbm, out_hbm,
          idx_v, scale_v, bufs, gsems, ssems):
    wid = lax.axis_index("s") * NC + lax.axis_index("c")
    base = wid * B_PER_W

    pltpu.sync_copy(idx_hbm.at[wid], idx_v)
    pltpu.sync_copy(scale_hbm, scale_v)
    svec = scale_v[...]

    def fire_gather(p, b):
        buf, sem = bufs[b], gsems[b]
        pltpu.async_copy(table_hbm.at[idx_v.at[2 * p]],
                         buf.at[pl.ds(0, C)], sem)
        pltpu.async_copy(table_hbm.at[idx_v.at[2 * p + 1]],
                         buf.at[pl.ds(C, C)], sem)

    def wait_gather(b):
        pltpu.make_async_copy(table_hbm.at[idx_v.at[0]],
                              bufs[b].at[pl.ds(0, C)], gsems[b]).wait()
        pltpu.make_async_copy(table_hbm.at[idx_v.at[0]],
                              bufs[b].at[pl.ds(C, C)], gsems[b]).wait()

    def scale_buf(b):
        buf = bufs[b]

        def group(i, _):
            r0 = i * 16
            for r in range(16):
                for k in range(D // L):
                    sl = pl.ds(k * L, L)
                    buf[r0 + r, sl] = buf[r0 + r, sl] * svec
            return 0
        lax.fori_loop(0, PAIR // 16, group, 0)

    def start_store(p, b):
        pltpu.async_copy(bufs[b], out_hbm.at[pl.ds(base + p * PAIR, PAIR)],
                         ssems[b])

    def wait_store(b):
        pltpu.make_async_copy(bufs[b],
                              out_hbm.at[pl.ds(base, PAIR)], ssems[b]).wait()

    for b in range(NBUF):
        fire_gather(b, b)

    def step(t, _):
        p0 = t * NBUF
        for b in range(NBUF):
            wait_gather(b)
            scale_buf(b)
            start_store(p0 + b, b)

        @pl.when(t < ROUNDS - 1)
        def _():
            for b in range(NBUF):
                wait_store(b)
                fire_gather(p0 + NBUF + b, b)
        return 0

    lax.fori_loop(0, ROUNDS, step, 0)
    for b in range(NBUF):
        wait_store(b)


@jax.jit
def _embed(table, idx, scale_vec):
    mesh = plsc.VectorSubcoreMesh(core_axis_name="c", subcore_axis_name="s")
    k = pl.kernel(
        _body,
        out_type=jax.ShapeDtypeStruct((B, D), jnp.float32),
        mesh=mesh,
        scratch_types=[
            pltpu.VMEM((NCHUNK, C), jnp.int32),
            pltpu.VMEM((L,), jnp.float32),
            [pltpu.VMEM((PAIR, D), jnp.float32) for _ in range(NBUF)],
            [pltpu.SemaphoreType.DMA for _ in range(NBUF)],
            [pltpu.SemaphoreType.DMA for _ in range(NBUF)],
        ],
        compiler_params=pltpu.CompilerParams(use_tc_tiling_on_sc=False),
    )
    return k(table, idx, scale_vec)


def kernel(token_ids, embed_weight, scale):
    orig_shape = token_ids.shape
    idx = token_ids.reshape(NW, NCHUNK, C).astype(jnp.int32)
    scale_vec = jnp.broadcast_to(scale.astype(jnp.float32), (L,))
    out = _embed(embed_weight, idx, scale_vec)
    return out.reshape(*orig_shape, D)
